# baseline (speedup 1.0000x reference)
import functools

import jax
import jax.numpy as jnp
from jax import lax
from jax.experimental import pallas as pl
from jax.experimental.pallas import tpu as pltpu
from jax.experimental.pallas import tpu_sc as plsc

_LAMBDA = 300.0
_NSEG = 64
_NTILES = 32
_LANES = 16
_CBLK = 2048
_SENTINEL = 1.0e6


def _huber(x):
    ax = jnp.abs(x)
    return jnp.where(ax < 1.0, 0.5 * x * x, ax - 0.5)


def _sc_body(pred_hbm, tgt_hbm, stats_hbm, compact_hbm, counts_hbm,
             ps0, ps1, ps2, ts0, ts1, ts2, h_cnt, h_p0, h_p1, h_p2, h_hub,
             cb0, cb1, cb2, stats_loc, cnt_loc, *, nbatch, hw):
    cols = hw // _NTILES
    nsub = cols // _CBLK
    nvec = _CBLK // _LANES
    wid = lax.axis_index("s") * 2 + lax.axis_index("c")
    lane = lax.broadcasted_iota(jnp.int32, (_LANES,), 0)
    onesf = jnp.ones((_LANES,), jnp.float32)
    zerof = jnp.zeros((_LANES,), jnp.float32)
    sentf = jnp.full((_LANES,), _SENTINEL, jnp.float32)

    for b in range(nbatch):
        def _zero(k, _):
            s = pl.ds(k * _LANES, _LANES)
            h_cnt[s] = zerof
            h_p0[s] = zerof
            h_p1[s] = zerof
            h_p2[s] = zerof
            h_hub[s] = zerof
            return 0

        lax.fori_loop(0, (_NSEG * _LANES) // _LANES, _zero, 0)

        def _fill(k, _):
            s = pl.ds(k * _LANES, _LANES)
            cb0[s] = sentf
            cb1[s] = sentf
            cb2[s] = sentf
            return 0

        lax.fori_loop(0, cols // _LANES, _fill, 0)

        off = jnp.int32(0)
        for sub in range(nsub):
            col0 = wid * cols + sub * _CBLK
            for c, (pst, tst) in enumerate([(ps0, ts0), (ps1, ts1),
                                            (ps2, ts2)]):
                pltpu.sync_copy(pred_hbm.at[3 * b + c, pl.ds(col0, _CBLK)],
                                pst)
                pltpu.sync_copy(tgt_hbm.at[3 * b + c, pl.ds(col0, _CBLK)],
                                tst)

            def _vec(i, off):
                s = pl.ds(i * _LANES, _LANES)
                t0 = ts0[s]
                t1 = ts1[s]
                t2 = ts2[s]
                idv = t0 * 16 + t1 * 4 + t2
                addr = lane * _NSEG + idv
                p0 = ps0[s]
                p1 = ps1[s]
                p2 = ps2[s]
                plsc.addupdate_scatter(h_cnt, [addr], onesf)
                plsc.addupdate_scatter(h_p0, [addr], p0)
                plsc.addupdate_scatter(h_p1, [addr], p1)
                plsc.addupdate_scatter(h_p2, [addr], p2)
                hz = _huber(p0) + _huber(p1) + _huber(p2)
                hf = (_huber(p0 - 255.0) + _huber(p1 - 255.0)
                      + _huber(p2 - 255.0))
                plsc.addupdate_scatter(h_hub, [addr],
                                       jnp.where(idv == 0, hz, hf))
                msk = idv == 0
                plsc.store_compressed(cb0.at[pl.ds(off, _LANES)], p0, mask=msk)
                plsc.store_compressed(cb1.at[pl.ds(off, _LANES)], p1, mask=msk)
                plsc.store_compressed(cb2.at[pl.ds(off, _LANES)], p2, mask=msk)
                npix = plsc.all_reduce_population_count(msk)
                return off + jnp.max(npix)

            off = lax.fori_loop(0, nvec, _vec, off)

        hists = [h_cnt, h_p0, h_p1, h_p2, h_hub]
        for st in range(5):
            for g in range(_NSEG // _LANES):
                acc = hists[st][pl.ds(g * _LANES, _LANES)]
                for l in range(1, _LANES):
                    acc = acc + hists[st][pl.ds(l * _NSEG + g * _LANES,
                                                _LANES)]
                stats_loc[pl.ds(st * _NSEG + g * _LANES, _LANES)] = acc
        pltpu.sync_copy(stats_loc, stats_hbm.at[wid, b])

        nch = lax.div(off + (_CBLK - 1), _CBLK)
        pc = nch * _CBLK
        cnt_loc[...] = jnp.where(lane == 0, jnp.full((_LANES,), pc,
                                                     jnp.int32), 0)
        pltpu.sync_copy(cnt_loc, counts_hbm.at[wid, b])

        def _wr(jc, _):
            src = pl.ds(jc * _CBLK, _CBLK)
            dst = pl.ds(wid * cols + jc * _CBLK, _CBLK)
            pltpu.sync_copy(cb0.at[src], compact_hbm.at[3 * b + 0, dst])
            pltpu.sync_copy(cb1.at[src], compact_hbm.at[3 * b + 1, dst])
            pltpu.sync_copy(cb2.at[src], compact_hbm.at[3 * b + 2, dst])
            return 0

        lax.fori_loop(0, nch, _wr, 0)


def _sc_stats_compact(pred2, tgt2, nbatch, hw):
    cols = hw // _NTILES
    mesh = plsc.VectorSubcoreMesh(core_axis_name="c", subcore_axis_name="s")
    k = functools.partial(
        pl.kernel,
        out_type=[
            jax.ShapeDtypeStruct((_NTILES, nbatch, 5 * _NSEG), jnp.float32),
            jax.ShapeDtypeStruct((3 * nbatch, hw), jnp.float32),
            jax.ShapeDtypeStruct((_NTILES, nbatch, _LANES), jnp.int32),
        ],
        mesh=mesh,
        compiler_params=pltpu.CompilerParams(needs_layout_passes=False),
        scratch_types=[
            pltpu.VMEM((_CBLK,), jnp.float32),
            pltpu.VMEM((_CBLK,), jnp.float32),
            pltpu.VMEM((_CBLK,), jnp.float32),
            pltpu.VMEM((_CBLK,), jnp.int32),
            pltpu.VMEM((_CBLK,), jnp.int32),
            pltpu.VMEM((_CBLK,), jnp.int32),
            pltpu.VMEM((_NSEG * _LANES,), jnp.float32),
            pltpu.VMEM((_NSEG * _LANES,), jnp.float32),
            pltpu.VMEM((_NSEG * _LANES,), jnp.float32),
            pltpu.VMEM((_NSEG * _LANES,), jnp.float32),
            pltpu.VMEM((_NSEG * _LANES,), jnp.float32),
            pltpu.VMEM((cols + _LANES,), jnp.float32),
            pltpu.VMEM((cols + _LANES,), jnp.float32),
            pltpu.VMEM((cols + _LANES,), jnp.float32),
            pltpu.VMEM((5 * _NSEG,), jnp.float32),
            pltpu.VMEM((_LANES,), jnp.int32),
        ],
    )(functools.partial(_sc_body, nbatch=nbatch, hw=hw))
    return k(pred2, tgt2)


def _tc_body(counts_ref, nobg_ref, stats_ref, pred_ref, comp_ref, out_ref,
             seg_ref, acc_ref, *, nbatch, hw, nrb):
    t = pl.program_id(0)
    j = pl.program_id(1)

    @pl.when(jnp.logical_and(t == 0, j == 0))
    def _init():
        acc_ref[...] = jnp.zeros_like(acc_ref)
        stats1 = lax.dot_general(
            jnp.ones((1, _NTILES), jnp.float32), stats_ref[...],
            (((1,), (0,)), ((), ())),
            preferred_element_type=jnp.float32)
        eye = (lax.broadcasted_iota(jnp.int32, (_NSEG, _NSEG), 0)
               == lax.broadcasted_iota(jnp.int32, (_NSEG, _NSEG), 1)
               ).astype(jnp.float32)
        for bb in range(nbatch):
            base = bb * 5 * _NSEG
            cols = []
            for st in range(5):
                row = stats1[:, base + st * _NSEG:base + (st + 1) * _NSEG]
                cols.append(lax.dot_general(
                    eye, row, (((1,), (1,)), ((), ())),
                    preferred_element_type=jnp.float32))
            cnt, sp0, sp1, sp2, hub = cols
            size_safe = jnp.maximum(cnt, 1.0)
            means = jnp.concatenate([sp0, sp1, sp2], axis=1) / size_safe
            mnorm = jnp.sum(means * means, axis=1, keepdims=True)
            seg_ref[bb] = jnp.concatenate(
                [means, mnorm, cnt, hub, jnp.zeros((_NSEG, 2), jnp.float32)],
                axis=1)

    pred_all = pred_ref[...]
    comp_all = comp_ref[...]

    for bb in range(nbatch):
        seg = seg_ref[bb]
        pred = pred_all[3 * bb:3 * bb + 3, :]

        m0 = seg[0:1, 0:3]
        mn0 = seg[0:1, 3:4]
        pn = jnp.sum(pred * pred, axis=0, keepdims=True)
        g0 = lax.dot_general(m0, pred, (((1,), (0,)), ((), ())),
                             preferred_element_type=jnp.float32)
        t0 = _LAMBDA / (1.0 + mn0 + pn - 2.0 * g0)
        acc_ref[bb, 0:1, 1:2] += jnp.sum(t0).reshape(1, 1)

        pc = counts_ref[bb, t]

        @pl.when(j * _CBLK < pc)
        def _compact():
            comp = comp_all[3 * bb:3 * bb + 3, :]
            means = seg[:, 0:3]
            mn = seg[:, 3:4]
            cpn = jnp.sum(comp * comp, axis=0, keepdims=True)
            g = lax.dot_general(means, comp, (((1,), (0,)), ((), ())),
                                preferred_element_type=jnp.float32)
            tm = _LAMBDA / (1.0 + mn + cpn - 2.0 * g)
            acc_ref[bb, :, 0:1] += jnp.sum(tm, axis=1, keepdims=True)

    @pl.when(jnp.logical_and(t == _NTILES - 1, j == nrb - 1))
    def _finalize():
        total = jnp.zeros((1, 1), dtype=jnp.float32)
        rowidx = lax.broadcasted_iota(jnp.int32, (_NSEG, 1), 0)
        for bb in range(nbatch):
            segb = seg_ref[bb]
            cnt = segb[:, 4:5]
            hub = segb[:, 5:6]
            S = acc_ref[bb, :, 0:1]
            rs0 = acc_ref[bb, 0:1, 1:2]
            nobg = nobg_ref[bb, 0]
            present = cnt > 0.0
            size_safe = jnp.maximum(cnt, 1.0)
            var_loss = hub / (size_safe * 3.0)
            w = 10.0 * lax.rsqrt(size_safe)
            cnt0 = cnt[0:1, :]
            bg_present = cnt0 > 0.0
            use0 = jnp.logical_and(bg_present, nobg == 0)
            n_non = float(hw) - cnt0
            sep0 = (rs0 - S[0:1, :]) / jnp.maximum(n_non, 1.0)
            contrib0 = (jnp.where(use0, var_loss[0:1, :], 0.0)
                        + jnp.where(jnp.logical_and(use0, n_non > 0.0),
                                    w[0:1, :] * sep0, 0.0))
            sepj = S / jnp.maximum(cnt0, 1.0)
            contribj = (jnp.where(present, var_loss, 0.0)
                        + jnp.where(jnp.logical_and(present, bg_present),
                                    w * sepj, 0.0))
            contrib = jnp.where(rowidx == 0, 0.0, contribj)
            loss_b = jnp.sum(contrib) + jnp.sum(contrib0)
            ctv = jnp.where(rowidx == 0,
                            jnp.broadcast_to(use0.astype(jnp.float32),
                                             (_NSEG, 1)),
                            present.astype(jnp.float32))
            ct = jnp.maximum(jnp.sum(ctv), 1.0)
            total += loss_b / ct
        out_ref[...] = total / float(nbatch)


def kernel(prediction, target, no_bg):
    prediction = prediction.astype(jnp.float32)
    B, C, H, W = prediction.shape
    HW = H * W
    cols = HW // _NTILES
    nrb = cols // _CBLK
    pred2 = prediction.reshape(B * C, HW)
    tgt2 = target.astype(jnp.int32).reshape(B * C, HW)

    stats_raw, compact, counts_raw = _sc_stats_compact(pred2, tgt2, B, HW)
    stats_flat = stats_raw.reshape(_NTILES, B * 5 * _NSEG)
    counts = counts_raw[:, :, 0].T
    nobg = no_bg.astype(jnp.int32).reshape(B, 1)

    out = pl.pallas_call(
        functools.partial(_tc_body, nbatch=B, hw=HW, nrb=nrb),
        grid=(_NTILES, nrb),
        in_specs=[
            pl.BlockSpec(memory_space=pltpu.SMEM),
            pl.BlockSpec(memory_space=pltpu.SMEM),
            pl.BlockSpec((_NTILES, B * 5 * _NSEG), lambda t, j: (0, 0)),
            pl.BlockSpec((B * C, _CBLK), lambda t, j: (0, t * nrb + j)),
            pl.BlockSpec((B * C, _CBLK), lambda t, j: (0, t * nrb + j)),
        ],
        out_specs=pl.BlockSpec((1, 1), lambda t, j: (0, 0)),
        out_shape=jax.ShapeDtypeStruct((1, 1), jnp.float32),
        scratch_shapes=[
            pltpu.VMEM((B, _NSEG, 8), jnp.float32),
            pltpu.VMEM((B, _NSEG, 8), jnp.float32),
        ],
    )(counts, nobg, stats_flat, pred2, compact)
    return out[0, 0]


# speedup vs baseline: 1.1137x; 1.1137x over previous
import functools

import jax
import jax.numpy as jnp
from jax import lax
from jax.experimental import pallas as pl
from jax.experimental.pallas import tpu as pltpu
from jax.experimental.pallas import tpu_sc as plsc

_LAMBDA = 300.0
_NSEG = 64
_NTILES = 32
_LANES = 16
_CBLK = 2048
_SENTINEL = 1.0e6


def _huber(x):
    ax = jnp.abs(x)
    return jnp.where(ax < 1.0, 0.5 * x * x, ax - 0.5)


def _sc_body(pred_hbm, tgt_hbm, stats_hbm, compact_hbm, counts_hbm,
             ps0, ps1, ps2, ts0, ts1, ts2, h_cnt, h_p0, h_p1, h_p2, h_hub,
             cb0, cb1, cb2, stats_loc, cnt_loc, *, nbatch, hw):
    cols = hw // _NTILES
    wid = lax.axis_index("s") * 2 + lax.axis_index("c")
    lane = lax.broadcasted_iota(jnp.int32, (_LANES,), 0)
    onesf = jnp.ones((_LANES,), jnp.float32)
    zerof = jnp.zeros((_LANES,), jnp.float32)
    sentf = jnp.full((_LANES,), _SENTINEL, jnp.float32)

    for b in range(nbatch):
        def _zero(k, _):
            s = pl.ds(k * _LANES, _LANES)
            h_cnt[s] = zerof
            h_p0[s] = zerof
            h_p1[s] = zerof
            h_p2[s] = zerof
            h_hub[s] = zerof
            return 0

        lax.fori_loop(0, (_NSEG * _LANES) // _LANES, _zero, 0)

        def _fill(k, _):
            s = pl.ds(k * _LANES, _LANES)
            cb0[s] = sentf
            cb1[s] = sentf
            cb2[s] = sentf
            return 0

        lax.fori_loop(0, cols // _LANES, _fill, 0)

        col0 = wid * cols
        for c, (pst, tst) in enumerate([(ps0, ts0), (ps1, ts1), (ps2, ts2)]):
            pltpu.sync_copy(pred_hbm.at[3 * b + c, pl.ds(col0, cols)], pst)
            pltpu.sync_copy(tgt_hbm.at[3 * b + c, pl.ds(col0, cols)], tst)

        def _vec(i, off):
            s = pl.ds(i * _LANES, _LANES)
            t0 = ts0[s]
            t1 = ts1[s]
            t2 = ts2[s]
            idv = t0 * 16 + t1 * 4 + t2
            addr = lane * _NSEG + idv
            p0 = ps0[s]
            p1 = ps1[s]
            p2 = ps2[s]
            plsc.addupdate_scatter(h_cnt, [addr], onesf)
            plsc.addupdate_scatter(h_p0, [addr], p0)
            plsc.addupdate_scatter(h_p1, [addr], p1)
            plsc.addupdate_scatter(h_p2, [addr], p2)
            hz = _huber(p0) + _huber(p1) + _huber(p2)
            hf = (_huber(p0 - 255.0) + _huber(p1 - 255.0)
                  + _huber(p2 - 255.0))
            plsc.addupdate_scatter(h_hub, [addr],
                                   jnp.where(idv == 0, hz, hf))
            msk = idv == 0
            plsc.store_compressed(cb0.at[pl.ds(off, _LANES)], p0, mask=msk)
            plsc.store_compressed(cb1.at[pl.ds(off, _LANES)], p1, mask=msk)
            plsc.store_compressed(cb2.at[pl.ds(off, _LANES)], p2, mask=msk)
            npix = plsc.all_reduce_population_count(msk)
            return off + jnp.max(npix)

        off = lax.fori_loop(0, cols // _LANES, _vec, jnp.int32(0))

        hists = [h_cnt, h_p0, h_p1, h_p2, h_hub]
        for st in range(5):
            for g in range(_NSEG // _LANES):
                acc = hists[st][pl.ds(g * _LANES, _LANES)]
                for l in range(1, _LANES):
                    acc = acc + hists[st][pl.ds(l * _NSEG + g * _LANES,
                                                _LANES)]
                stats_loc[pl.ds(st * _NSEG + g * _LANES, _LANES)] = acc
        pltpu.sync_copy(stats_loc, stats_hbm.at[wid, b])

        nch = lax.div(off + (_CBLK - 1), _CBLK)
        pc = nch * _CBLK
        cnt_loc[...] = jnp.where(lane == 0, jnp.full((_LANES,), pc,
                                                     jnp.int32), 0)
        pltpu.sync_copy(cnt_loc, counts_hbm.at[wid, b])

        def _wr(jc, _):
            src = pl.ds(jc * _CBLK, _CBLK)
            dst = pl.ds(wid * cols + jc * _CBLK, _CBLK)
            pltpu.sync_copy(cb0.at[src], compact_hbm.at[3 * b + 0, dst])
            pltpu.sync_copy(cb1.at[src], compact_hbm.at[3 * b + 1, dst])
            pltpu.sync_copy(cb2.at[src], compact_hbm.at[3 * b + 2, dst])
            return 0

        lax.fori_loop(0, nch, _wr, 0)


def _sc_stats_compact(pred2, tgt2, nbatch, hw):
    cols = hw // _NTILES
    mesh = plsc.VectorSubcoreMesh(core_axis_name="c", subcore_axis_name="s")
    k = functools.partial(
        pl.kernel,
        out_type=[
            jax.ShapeDtypeStruct((_NTILES, nbatch, 5 * _NSEG), jnp.float32),
            jax.ShapeDtypeStruct((3 * nbatch, hw), jnp.float32),
            jax.ShapeDtypeStruct((_NTILES, nbatch, _LANES), jnp.int32),
        ],
        mesh=mesh,
        compiler_params=pltpu.CompilerParams(needs_layout_passes=False),
        scratch_types=[
            pltpu.VMEM((cols,), jnp.float32),
            pltpu.VMEM((cols,), jnp.float32),
            pltpu.VMEM((cols,), jnp.float32),
            pltpu.VMEM((cols,), jnp.int32),
            pltpu.VMEM((cols,), jnp.int32),
            pltpu.VMEM((cols,), jnp.int32),
            pltpu.VMEM((_NSEG * _LANES,), jnp.float32),
            pltpu.VMEM((_NSEG * _LANES,), jnp.float32),
            pltpu.VMEM((_NSEG * _LANES,), jnp.float32),
            pltpu.VMEM((_NSEG * _LANES,), jnp.float32),
            pltpu.VMEM((_NSEG * _LANES,), jnp.float32),
            pltpu.VMEM((cols + _LANES,), jnp.float32),
            pltpu.VMEM((cols + _LANES,), jnp.float32),
            pltpu.VMEM((cols + _LANES,), jnp.float32),
            pltpu.VMEM((5 * _NSEG,), jnp.float32),
            pltpu.VMEM((_LANES,), jnp.int32),
        ],
    )(functools.partial(_sc_body, nbatch=nbatch, hw=hw))
    return k(pred2, tgt2)


def _tc_body(counts_ref, nobg_ref, stats_ref, pred_ref, comp_ref, out_ref,
             seg_ref, acc_ref, *, nbatch, hw, nrb):
    t = pl.program_id(0)
    j = pl.program_id(1)

    @pl.when(jnp.logical_and(t == 0, j == 0))
    def _init():
        acc_ref[...] = jnp.zeros_like(acc_ref)
        stats1 = lax.dot_general(
            jnp.ones((1, _NTILES), jnp.float32), stats_ref[...],
            (((1,), (0,)), ((), ())),
            preferred_element_type=jnp.float32)
        eye = (lax.broadcasted_iota(jnp.int32, (_NSEG, _NSEG), 0)
               == lax.broadcasted_iota(jnp.int32, (_NSEG, _NSEG), 1)
               ).astype(jnp.float32)
        for bb in range(nbatch):
            base = bb * 5 * _NSEG
            cols = []
            for st in range(5):
                row = stats1[:, base + st * _NSEG:base + (st + 1) * _NSEG]
                cols.append(lax.dot_general(
                    eye, row, (((1,), (1,)), ((), ())),
                    preferred_element_type=jnp.float32))
            cnt, sp0, sp1, sp2, hub = cols
            size_safe = jnp.maximum(cnt, 1.0)
            means = jnp.concatenate([sp0, sp1, sp2], axis=1) / size_safe
            mnorm = jnp.sum(means * means, axis=1, keepdims=True)
            seg_ref[bb] = jnp.concatenate(
                [means, mnorm, cnt, hub, jnp.zeros((_NSEG, 2), jnp.float32)],
                axis=1)

    pred_all = pred_ref[...]
    comp_all = comp_ref[...]

    for bb in range(nbatch):
        seg = seg_ref[bb]
        pred = pred_all[3 * bb:3 * bb + 3, :]

        m0 = seg[0:1, 0:3]
        mn0 = seg[0:1, 3:4]
        pn = jnp.sum(pred * pred, axis=0, keepdims=True)
        g0 = lax.dot_general(m0, pred, (((1,), (0,)), ((), ())),
                             preferred_element_type=jnp.float32)
        t0 = _LAMBDA / (1.0 + mn0 + pn - 2.0 * g0)
        acc_ref[bb, 0:1, 1:2] += jnp.sum(t0).reshape(1, 1)

        pc = counts_ref[bb, t]

        @pl.when(j * _CBLK < pc)
        def _compact():
            comp = comp_all[3 * bb:3 * bb + 3, :]
            means = seg[:, 0:3]
            mn = seg[:, 3:4]
            cpn = jnp.sum(comp * comp, axis=0, keepdims=True)
            g = lax.dot_general(means, comp, (((1,), (0,)), ((), ())),
                                preferred_element_type=jnp.float32)
            tm = _LAMBDA / (1.0 + mn + cpn - 2.0 * g)
            acc_ref[bb, :, 0:1] += jnp.sum(tm, axis=1, keepdims=True)

    @pl.when(jnp.logical_and(t == _NTILES - 1, j == nrb - 1))
    def _finalize():
        total = jnp.zeros((1, 1), dtype=jnp.float32)
        rowidx = lax.broadcasted_iota(jnp.int32, (_NSEG, 1), 0)
        for bb in range(nbatch):
            segb = seg_ref[bb]
            cnt = segb[:, 4:5]
            hub = segb[:, 5:6]
            S = acc_ref[bb, :, 0:1]
            rs0 = acc_ref[bb, 0:1, 1:2]
            nobg = nobg_ref[bb, 0]
            present = cnt > 0.0
            size_safe = jnp.maximum(cnt, 1.0)
            var_loss = hub / (size_safe * 3.0)
            w = 10.0 * lax.rsqrt(size_safe)
            cnt0 = cnt[0:1, :]
            bg_present = cnt0 > 0.0
            use0 = jnp.logical_and(bg_present, nobg == 0)
            n_non = float(hw) - cnt0
            sep0 = (rs0 - S[0:1, :]) / jnp.maximum(n_non, 1.0)
            contrib0 = (jnp.where(use0, var_loss[0:1, :], 0.0)
                        + jnp.where(jnp.logical_and(use0, n_non > 0.0),
                                    w[0:1, :] * sep0, 0.0))
            sepj = S / jnp.maximum(cnt0, 1.0)
            contribj = (jnp.where(present, var_loss, 0.0)
                        + jnp.where(jnp.logical_and(present, bg_present),
                                    w * sepj, 0.0))
            contrib = jnp.where(rowidx == 0, 0.0, contribj)
            loss_b = jnp.sum(contrib) + jnp.sum(contrib0)
            ctv = jnp.where(rowidx == 0,
                            jnp.broadcast_to(use0.astype(jnp.float32),
                                             (_NSEG, 1)),
                            present.astype(jnp.float32))
            ct = jnp.maximum(jnp.sum(ctv), 1.0)
            total += loss_b / ct
        out_ref[...] = total / float(nbatch)


def kernel(prediction, target, no_bg):
    prediction = prediction.astype(jnp.float32)
    B, C, H, W = prediction.shape
    HW = H * W
    cols = HW // _NTILES
    nrb = cols // _CBLK
    pred2 = prediction.reshape(B * C, HW)
    tgt2 = target.astype(jnp.int32).reshape(B * C, HW)

    stats_raw, compact, counts_raw = _sc_stats_compact(pred2, tgt2, B, HW)
    stats_flat = stats_raw.reshape(_NTILES, B * 5 * _NSEG)
    counts = counts_raw[:, :, 0].T
    nobg = no_bg.astype(jnp.int32).reshape(B, 1)

    out = pl.pallas_call(
        functools.partial(_tc_body, nbatch=B, hw=HW, nrb=nrb),
        grid=(_NTILES, nrb),
        in_specs=[
            pl.BlockSpec(memory_space=pltpu.SMEM),
            pl.BlockSpec(memory_space=pltpu.SMEM),
            pl.BlockSpec((_NTILES, B * 5 * _NSEG), lambda t, j: (0, 0)),
            pl.BlockSpec((B * C, _CBLK), lambda t, j: (0, t * nrb + j)),
            pl.BlockSpec((B * C, _CBLK), lambda t, j: (0, t * nrb + j)),
        ],
        out_specs=pl.BlockSpec((1, 1), lambda t, j: (0, 0)),
        out_shape=jax.ShapeDtypeStruct((1, 1), jnp.float32),
        scratch_shapes=[
            pltpu.VMEM((B, _NSEG, 8), jnp.float32),
            pltpu.VMEM((B, _NSEG, 8), jnp.float32),
        ],
    )(counts, nobg, stats_flat, pred2, compact)
    return out[0, 0]


# speedup vs baseline: 1.8173x; 1.6318x over previous
import functools

import jax
import jax.numpy as jnp
from jax import lax
from jax.experimental import pallas as pl
from jax.experimental.pallas import tpu as pltpu
from jax.experimental.pallas import tpu_sc as plsc

_LAMBDA = 300.0
_NSEG = 64
_NTILES = 32
_LANES = 16
_CBLK = 2048
_SENTINEL = 1.0e6


def _huber(x):
    ax = jnp.abs(x)
    return jnp.where(ax < 1.0, 0.5 * x * x, ax - 0.5)


def _sc_body(pred_hbm, tgt_hbm, stats_hbm, compact_hbm, counts_hbm,
             ps0, ps1, ps2, ts0, ts1, ts2, h_cnt, h_p0, h_p1, h_p2, h_hub,
             cb0, cb1, cb2, stats_loc, cnt_loc, *, nbatch, hw):
    cols = hw // _NTILES
    wid = lax.axis_index("s") * 2 + lax.axis_index("c")
    lane = lax.broadcasted_iota(jnp.int32, (_LANES,), 0)
    onesf = jnp.ones((_LANES,), jnp.float32)
    zerof = jnp.zeros((_LANES,), jnp.float32)
    sentf = jnp.full((_LANES,), _SENTINEL, jnp.float32)

    for b in range(nbatch):
        def _zero(k, _):
            s = pl.ds(k * _LANES, _LANES)
            h_cnt[s] = zerof
            h_p0[s] = zerof
            h_p1[s] = zerof
            h_p2[s] = zerof
            h_hub[s] = zerof
            return 0

        lax.fori_loop(0, (_NSEG * _LANES) // _LANES, _zero, 0)

        def _fill(k, _):
            s = pl.ds(k * _LANES, _LANES)
            cb0[s] = sentf
            cb1[s] = sentf
            cb2[s] = sentf
            return 0

        lax.fori_loop(0, cols // _LANES, _fill, 0)

        col0 = wid * cols
        for c, (pst, tst) in enumerate([(ps0, ts0), (ps1, ts1), (ps2, ts2)]):
            pltpu.sync_copy(pred_hbm.at[3 * b + c, pl.ds(col0, cols)], pst)
            pltpu.sync_copy(tgt_hbm.at[3 * b + c, pl.ds(col0, cols)], tst)

        def _vec(i, off):
            s = pl.ds(i * _LANES, _LANES)
            t0 = ts0[s]
            t1 = ts1[s]
            t2 = ts2[s]
            idv = t0 * 16 + t1 * 4 + t2
            addr = lane * _NSEG + idv
            p0 = ps0[s]
            p1 = ps1[s]
            p2 = ps2[s]
            plsc.addupdate_scatter(h_cnt, [addr], onesf)
            plsc.addupdate_scatter(h_p0, [addr], p0)
            plsc.addupdate_scatter(h_p1, [addr], p1)
            plsc.addupdate_scatter(h_p2, [addr], p2)
            hz = _huber(p0) + _huber(p1) + _huber(p2)
            hf = (_huber(p0 - 255.0) + _huber(p1 - 255.0)
                  + _huber(p2 - 255.0))
            plsc.addupdate_scatter(h_hub, [addr],
                                   jnp.where(idv == 0, hz, hf))
            msk = idv == 0
            plsc.store_compressed(cb0.at[pl.ds(off, _LANES)], p0, mask=msk)
            plsc.store_compressed(cb1.at[pl.ds(off, _LANES)], p1, mask=msk)
            plsc.store_compressed(cb2.at[pl.ds(off, _LANES)], p2, mask=msk)
            npix = plsc.all_reduce_population_count(msk)
            return off + jnp.max(npix)

        off = lax.fori_loop(0, cols // _LANES, _vec, jnp.int32(0))

        hists = [h_cnt, h_p0, h_p1, h_p2, h_hub]
        for st in range(5):
            for g in range(_NSEG // _LANES):
                acc = hists[st][pl.ds(g * _LANES, _LANES)]
                for l in range(1, _LANES):
                    acc = acc + hists[st][pl.ds(l * _NSEG + g * _LANES,
                                                _LANES)]
                stats_loc[pl.ds(st * _NSEG + g * _LANES, _LANES)] = acc
        pltpu.sync_copy(stats_loc, stats_hbm.at[wid, b])

        nch = lax.div(off + (_CBLK - 1), _CBLK)
        pc = nch * _CBLK
        cnt_loc[...] = jnp.where(lane == 0, jnp.full((_LANES,), pc,
                                                     jnp.int32), 0)
        pltpu.sync_copy(cnt_loc, counts_hbm.at[wid, b])

        def _wr(jc, _):
            src = pl.ds(jc * _CBLK, _CBLK)
            dst = pl.ds(wid * cols + jc * _CBLK, _CBLK)
            pltpu.sync_copy(cb0.at[src], compact_hbm.at[3 * b + 0, dst])
            pltpu.sync_copy(cb1.at[src], compact_hbm.at[3 * b + 1, dst])
            pltpu.sync_copy(cb2.at[src], compact_hbm.at[3 * b + 2, dst])
            return 0

        lax.fori_loop(0, nch, _wr, 0)


def _sc_stats_compact(pred2, tgt2, nbatch, hw):
    cols = hw // _NTILES
    mesh = plsc.VectorSubcoreMesh(core_axis_name="c", subcore_axis_name="s")
    k = functools.partial(
        pl.kernel,
        out_type=[
            jax.ShapeDtypeStruct((_NTILES, nbatch, 5 * _NSEG), jnp.float32),
            jax.ShapeDtypeStruct((3 * nbatch, hw), jnp.float32),
            jax.ShapeDtypeStruct((_NTILES, nbatch, _LANES), jnp.int32),
        ],
        mesh=mesh,
        compiler_params=pltpu.CompilerParams(needs_layout_passes=False),
        scratch_types=[
            pltpu.VMEM((cols,), jnp.float32),
            pltpu.VMEM((cols,), jnp.float32),
            pltpu.VMEM((cols,), jnp.float32),
            pltpu.VMEM((cols,), jnp.int32),
            pltpu.VMEM((cols,), jnp.int32),
            pltpu.VMEM((cols,), jnp.int32),
            pltpu.VMEM((_NSEG * _LANES,), jnp.float32),
            pltpu.VMEM((_NSEG * _LANES,), jnp.float32),
            pltpu.VMEM((_NSEG * _LANES,), jnp.float32),
            pltpu.VMEM((_NSEG * _LANES,), jnp.float32),
            pltpu.VMEM((_NSEG * _LANES,), jnp.float32),
            pltpu.VMEM((cols + _LANES,), jnp.float32),
            pltpu.VMEM((cols + _LANES,), jnp.float32),
            pltpu.VMEM((cols + _LANES,), jnp.float32),
            pltpu.VMEM((5 * _NSEG,), jnp.float32),
            pltpu.VMEM((_LANES,), jnp.int32),
        ],
    )(functools.partial(_sc_body, nbatch=nbatch, hw=hw))
    return k(pred2, tgt2)


def _tc_body(counts_ref, nobg_ref, stats_ref, pred_ref, comp_ref, out_ref,
             seg_ref, acc_ref, *, nbatch, hw, nrb, tiles_per_step, nsteps):
    g = pl.program_id(0)
    cols = hw // _NTILES

    @pl.when(g == 0)
    def _init():
        acc_ref[...] = jnp.zeros_like(acc_ref)
        stats1 = lax.dot_general(
            jnp.ones((1, _NTILES), jnp.float32), stats_ref[...],
            (((1,), (0,)), ((), ())),
            preferred_element_type=jnp.float32)
        eye = (lax.broadcasted_iota(jnp.int32, (_NSEG, _NSEG), 0)
               == lax.broadcasted_iota(jnp.int32, (_NSEG, _NSEG), 1)
               ).astype(jnp.float32)
        for bb in range(nbatch):
            base = bb * 5 * _NSEG
            cols = []
            for st in range(5):
                row = stats1[:, base + st * _NSEG:base + (st + 1) * _NSEG]
                cols.append(lax.dot_general(
                    eye, row, (((1,), (1,)), ((), ())),
                    preferred_element_type=jnp.float32))
            cnt, sp0, sp1, sp2, hub = cols
            size_safe = jnp.maximum(cnt, 1.0)
            means = jnp.concatenate([sp0, sp1, sp2], axis=1) / size_safe
            mnorm = jnp.sum(means * means, axis=1, keepdims=True)
            seg_ref[bb] = jnp.concatenate(
                [means, mnorm, cnt, hub, jnp.zeros((_NSEG, 2), jnp.float32)],
                axis=1)

    pred_all = pred_ref[...]

    for bb in range(nbatch):
        seg = seg_ref[bb]
        pred = pred_all[3 * bb:3 * bb + 3, :]

        m0 = seg[0:1, 0:3]
        mn0 = seg[0:1, 3:4]
        pn = jnp.sum(pred * pred, axis=0, keepdims=True)
        g0 = lax.dot_general(m0, pred, (((1,), (0,)), ((), ())),
                             preferred_element_type=jnp.float32)
        t0 = _LAMBDA / (1.0 + mn0 + pn - 2.0 * g0)
        acc_ref[bb, 0:1, 1:2] += jnp.sum(t0).reshape(1, 1)

        means = seg[:, 0:3]
        mn = seg[:, 3:4]
        for r in range(tiles_per_step):
            pc = counts_ref[bb, g * tiles_per_step + r]
            base = r * cols

            def _chunk(jc, _, bb=bb, means=means, mn=mn, base=base):
                comp = comp_ref[pl.ds(3 * bb, 3),
                                pl.ds(base + jc * _CBLK, _CBLK)]
                cpn = jnp.sum(comp * comp, axis=0, keepdims=True)
                gg = lax.dot_general(means, comp, (((1,), (0,)), ((), ())),
                                     preferred_element_type=jnp.float32)
                tm = _LAMBDA / (1.0 + mn + cpn - 2.0 * gg)
                acc_ref[bb, :, 0:1] += jnp.sum(tm, axis=1, keepdims=True)
                return 0

            lax.fori_loop(0, pc // _CBLK, _chunk, 0)

    @pl.when(g == nsteps - 1)
    def _finalize():
        total = jnp.zeros((1, 1), dtype=jnp.float32)
        rowidx = lax.broadcasted_iota(jnp.int32, (_NSEG, 1), 0)
        for bb in range(nbatch):
            segb = seg_ref[bb]
            cnt = segb[:, 4:5]
            hub = segb[:, 5:6]
            S = acc_ref[bb, :, 0:1]
            rs0 = acc_ref[bb, 0:1, 1:2]
            nobg = nobg_ref[bb, 0]
            present = cnt > 0.0
            size_safe = jnp.maximum(cnt, 1.0)
            var_loss = hub / (size_safe * 3.0)
            w = 10.0 * lax.rsqrt(size_safe)
            cnt0 = cnt[0:1, :]
            bg_present = cnt0 > 0.0
            use0 = jnp.logical_and(bg_present, nobg == 0)
            n_non = float(hw) - cnt0
            sep0 = (rs0 - S[0:1, :]) / jnp.maximum(n_non, 1.0)
            contrib0 = (jnp.where(use0, var_loss[0:1, :], 0.0)
                        + jnp.where(jnp.logical_and(use0, n_non > 0.0),
                                    w[0:1, :] * sep0, 0.0))
            sepj = S / jnp.maximum(cnt0, 1.0)
            contribj = (jnp.where(present, var_loss, 0.0)
                        + jnp.where(jnp.logical_and(present, bg_present),
                                    w * sepj, 0.0))
            contrib = jnp.where(rowidx == 0, 0.0, contribj)
            loss_b = jnp.sum(contrib) + jnp.sum(contrib0)
            ctv = jnp.where(rowidx == 0,
                            jnp.broadcast_to(use0.astype(jnp.float32),
                                             (_NSEG, 1)),
                            present.astype(jnp.float32))
            ct = jnp.maximum(jnp.sum(ctv), 1.0)
            total += loss_b / ct
        out_ref[...] = total / float(nbatch)


def kernel(prediction, target, no_bg):
    prediction = prediction.astype(jnp.float32)
    B, C, H, W = prediction.shape
    HW = H * W
    cols = HW // _NTILES
    nrb = cols // _CBLK
    pred2 = prediction.reshape(B * C, HW)
    tgt2 = target.astype(jnp.int32).reshape(B * C, HW)

    stats_raw, compact, counts_raw = _sc_stats_compact(pred2, tgt2, B, HW)
    stats_flat = stats_raw.reshape(_NTILES, B * 5 * _NSEG)
    counts = counts_raw[:, :, 0].T
    nobg = no_bg.astype(jnp.int32).reshape(B, 1)

    tiles_per_step = 2
    nsteps = _NTILES // tiles_per_step
    span = tiles_per_step * (HW // _NTILES)
    out = pl.pallas_call(
        functools.partial(_tc_body, nbatch=B, hw=HW, nrb=nrb,
                          tiles_per_step=tiles_per_step, nsteps=nsteps),
        grid=(nsteps,),
        in_specs=[
            pl.BlockSpec(memory_space=pltpu.SMEM),
            pl.BlockSpec(memory_space=pltpu.SMEM),
            pl.BlockSpec((_NTILES, B * 5 * _NSEG), lambda g: (0, 0)),
            pl.BlockSpec((B * C, span), lambda g: (0, g)),
            pl.BlockSpec((B * C, span), lambda g: (0, g)),
        ],
        out_specs=pl.BlockSpec((1, 1), lambda g: (0, 0)),
        out_shape=jax.ShapeDtypeStruct((1, 1), jnp.float32),
        scratch_shapes=[
            pltpu.VMEM((B, _NSEG, 8), jnp.float32),
            pltpu.VMEM((B, _NSEG, 8), jnp.float32),
        ],
    )(counts, nobg, stats_flat, pred2, compact)
    return out[0, 0]


# speedup vs baseline: 1.9094x; 1.0506x over previous
import functools

import jax
import jax.numpy as jnp
from jax import lax
from jax.experimental import pallas as pl
from jax.experimental.pallas import tpu as pltpu
from jax.experimental.pallas import tpu_sc as plsc

_LAMBDA = 300.0
_NSEG = 64
_NTILES = 32
_LANES = 16
_CBLK = 512
_SENTINEL = 1.0e6


def _huber(x):
    ax = jnp.abs(x)
    return jnp.where(ax < 1.0, 0.5 * x * x, ax - 0.5)


def _sc_body(pred_hbm, tgt_hbm, stats_hbm, compact_hbm, counts_hbm,
             ps0, ps1, ps2, ts0, ts1, ts2, h_cnt, h_p0, h_p1, h_p2, h_hub,
             cb0, cb1, cb2, stats_loc, cnt_loc, *, nbatch, hw):
    cols = hw // _NTILES
    wid = lax.axis_index("s") * 2 + lax.axis_index("c")
    lane = lax.broadcasted_iota(jnp.int32, (_LANES,), 0)
    onesf = jnp.ones((_LANES,), jnp.float32)
    zerof = jnp.zeros((_LANES,), jnp.float32)
    sentf = jnp.full((_LANES,), _SENTINEL, jnp.float32)

    for b in range(nbatch):
        def _zero(k, _):
            s = pl.ds(k * _LANES, _LANES)
            h_cnt[s] = zerof
            h_p0[s] = zerof
            h_p1[s] = zerof
            h_p2[s] = zerof
            h_hub[s] = zerof
            return 0

        lax.fori_loop(0, (_NSEG * _LANES) // _LANES, _zero, 0)

        def _fill(k, _):
            s = pl.ds(k * _LANES, _LANES)
            cb0[s] = sentf
            cb1[s] = sentf
            cb2[s] = sentf
            return 0

        lax.fori_loop(0, cols // _LANES, _fill, 0)

        col0 = wid * cols
        for c, (pst, tst) in enumerate([(ps0, ts0), (ps1, ts1), (ps2, ts2)]):
            pltpu.sync_copy(pred_hbm.at[3 * b + c, pl.ds(col0, cols)], pst)
            pltpu.sync_copy(tgt_hbm.at[3 * b + c, pl.ds(col0, cols)], tst)

        def _vec(i, off):
            s = pl.ds(i * _LANES, _LANES)
            t0 = ts0[s]
            t1 = ts1[s]
            t2 = ts2[s]
            idv = t0 * 16 + t1 * 4 + t2
            addr = lane * _NSEG + idv
            p0 = ps0[s]
            p1 = ps1[s]
            p2 = ps2[s]
            plsc.addupdate_scatter(h_cnt, [addr], onesf)
            plsc.addupdate_scatter(h_p0, [addr], p0)
            plsc.addupdate_scatter(h_p1, [addr], p1)
            plsc.addupdate_scatter(h_p2, [addr], p2)
            hz = _huber(p0) + _huber(p1) + _huber(p2)
            hf = (_huber(p0 - 255.0) + _huber(p1 - 255.0)
                  + _huber(p2 - 255.0))
            plsc.addupdate_scatter(h_hub, [addr],
                                   jnp.where(idv == 0, hz, hf))
            msk = idv == 0
            plsc.store_compressed(cb0.at[pl.ds(off, _LANES)], p0, mask=msk)
            plsc.store_compressed(cb1.at[pl.ds(off, _LANES)], p1, mask=msk)
            plsc.store_compressed(cb2.at[pl.ds(off, _LANES)], p2, mask=msk)
            npix = plsc.all_reduce_population_count(msk)
            return off + jnp.max(npix)

        off = lax.fori_loop(0, cols // _LANES, _vec, jnp.int32(0))

        hists = [h_cnt, h_p0, h_p1, h_p2, h_hub]
        for st in range(5):
            for g in range(_NSEG // _LANES):
                acc = hists[st][pl.ds(g * _LANES, _LANES)]
                for l in range(1, _LANES):
                    acc = acc + hists[st][pl.ds(l * _NSEG + g * _LANES,
                                                _LANES)]
                stats_loc[pl.ds(st * _NSEG + g * _LANES, _LANES)] = acc
        pltpu.sync_copy(stats_loc, stats_hbm.at[wid, b])

        nch = lax.div(off + (_CBLK - 1), _CBLK)
        pc = nch * _CBLK
        cnt_loc[...] = jnp.where(lane == 0, jnp.full((_LANES,), pc,
                                                     jnp.int32), 0)
        pltpu.sync_copy(cnt_loc, counts_hbm.at[wid, b])

        def _wr(jc, _):
            src = pl.ds(jc * _CBLK, _CBLK)
            dst = pl.ds(wid * cols + jc * _CBLK, _CBLK)
            pltpu.sync_copy(cb0.at[src], compact_hbm.at[3 * b + 0, dst])
            pltpu.sync_copy(cb1.at[src], compact_hbm.at[3 * b + 1, dst])
            pltpu.sync_copy(cb2.at[src], compact_hbm.at[3 * b + 2, dst])
            return 0

        lax.fori_loop(0, nch, _wr, 0)


def _sc_stats_compact(pred2, tgt2, nbatch, hw):
    cols = hw // _NTILES
    mesh = plsc.VectorSubcoreMesh(core_axis_name="c", subcore_axis_name="s")
    k = functools.partial(
        pl.kernel,
        out_type=[
            jax.ShapeDtypeStruct((_NTILES, nbatch, 5 * _NSEG), jnp.float32),
            jax.ShapeDtypeStruct((3 * nbatch, hw), jnp.float32),
            jax.ShapeDtypeStruct((_NTILES, nbatch, _LANES), jnp.int32),
        ],
        mesh=mesh,
        compiler_params=pltpu.CompilerParams(needs_layout_passes=False),
        scratch_types=[
            pltpu.VMEM((cols,), jnp.float32),
            pltpu.VMEM((cols,), jnp.float32),
            pltpu.VMEM((cols,), jnp.float32),
            pltpu.VMEM((cols,), jnp.int32),
            pltpu.VMEM((cols,), jnp.int32),
            pltpu.VMEM((cols,), jnp.int32),
            pltpu.VMEM((_NSEG * _LANES,), jnp.float32),
            pltpu.VMEM((_NSEG * _LANES,), jnp.float32),
            pltpu.VMEM((_NSEG * _LANES,), jnp.float32),
            pltpu.VMEM((_NSEG * _LANES,), jnp.float32),
            pltpu.VMEM((_NSEG * _LANES,), jnp.float32),
            pltpu.VMEM((cols + _LANES,), jnp.float32),
            pltpu.VMEM((cols + _LANES,), jnp.float32),
            pltpu.VMEM((cols + _LANES,), jnp.float32),
            pltpu.VMEM((5 * _NSEG,), jnp.float32),
            pltpu.VMEM((_LANES,), jnp.int32),
        ],
    )(functools.partial(_sc_body, nbatch=nbatch, hw=hw))
    return k(pred2, tgt2)


def _tc_body(counts_ref, nobg_ref, stats_ref, pred_ref, comp_ref, out_ref,
             seg_ref, acc_ref, *, nbatch, hw, nrb, tiles_per_step, nsteps):
    g = pl.program_id(0)
    cols = hw // _NTILES

    @pl.when(g == 0)
    def _init():
        acc_ref[...] = jnp.zeros_like(acc_ref)
        stats1 = lax.dot_general(
            jnp.ones((1, _NTILES), jnp.float32), stats_ref[...],
            (((1,), (0,)), ((), ())),
            preferred_element_type=jnp.float32)
        eye = (lax.broadcasted_iota(jnp.int32, (_NSEG, _NSEG), 0)
               == lax.broadcasted_iota(jnp.int32, (_NSEG, _NSEG), 1)
               ).astype(jnp.float32)
        for bb in range(nbatch):
            base = bb * 5 * _NSEG
            cols = []
            for st in range(5):
                row = stats1[:, base + st * _NSEG:base + (st + 1) * _NSEG]
                cols.append(lax.dot_general(
                    eye, row, (((1,), (1,)), ((), ())),
                    preferred_element_type=jnp.float32))
            cnt, sp0, sp1, sp2, hub = cols
            size_safe = jnp.maximum(cnt, 1.0)
            means = jnp.concatenate([sp0, sp1, sp2], axis=1) / size_safe
            mnorm = jnp.sum(means * means, axis=1, keepdims=True)
            seg_ref[bb] = jnp.concatenate(
                [means, mnorm, cnt, hub, jnp.zeros((_NSEG, 2), jnp.float32)],
                axis=1)

    pred_all = pred_ref[...]

    for bb in range(nbatch):
        seg = seg_ref[bb]
        pred = pred_all[3 * bb:3 * bb + 3, :]

        m0 = seg[0:1, 0:3]
        mn0 = seg[0:1, 3:4]
        pn = jnp.sum(pred * pred, axis=0, keepdims=True)
        g0 = lax.dot_general(m0, pred, (((1,), (0,)), ((), ())),
                             preferred_element_type=jnp.float32)
        t0 = _LAMBDA / (1.0 + mn0 + pn - 2.0 * g0)
        acc_ref[bb, 0:1, 1:2] += jnp.sum(t0).reshape(1, 1)

        means = seg[:, 0:3]
        mn = seg[:, 3:4]
        for r in range(tiles_per_step):
            pc = counts_ref[bb, g * tiles_per_step + r]
            base = r * cols

            def _chunk(jc, _, bb=bb, means=means, mn=mn, base=base):
                comp = comp_ref[pl.ds(3 * bb, 3),
                                pl.ds(base + jc * _CBLK, _CBLK)]
                cpn = jnp.sum(comp * comp, axis=0, keepdims=True)
                gg = lax.dot_general(means, comp, (((1,), (0,)), ((), ())),
                                     preferred_element_type=jnp.float32)
                tm = _LAMBDA / (1.0 + mn + cpn - 2.0 * gg)
                acc_ref[bb, :, 0:1] += jnp.sum(tm, axis=1, keepdims=True)
                return 0

            lax.fori_loop(0, pc // _CBLK, _chunk, 0)

    @pl.when(g == nsteps - 1)
    def _finalize():
        total = jnp.zeros((1, 1), dtype=jnp.float32)
        rowidx = lax.broadcasted_iota(jnp.int32, (_NSEG, 1), 0)
        for bb in range(nbatch):
            segb = seg_ref[bb]
            cnt = segb[:, 4:5]
            hub = segb[:, 5:6]
            S = acc_ref[bb, :, 0:1]
            rs0 = acc_ref[bb, 0:1, 1:2]
            nobg = nobg_ref[bb, 0]
            present = cnt > 0.0
            size_safe = jnp.maximum(cnt, 1.0)
            var_loss = hub / (size_safe * 3.0)
            w = 10.0 * lax.rsqrt(size_safe)
            cnt0 = cnt[0:1, :]
            bg_present = cnt0 > 0.0
            use0 = jnp.logical_and(bg_present, nobg == 0)
            n_non = float(hw) - cnt0
            sep0 = (rs0 - S[0:1, :]) / jnp.maximum(n_non, 1.0)
            contrib0 = (jnp.where(use0, var_loss[0:1, :], 0.0)
                        + jnp.where(jnp.logical_and(use0, n_non > 0.0),
                                    w[0:1, :] * sep0, 0.0))
            sepj = S / jnp.maximum(cnt0, 1.0)
            contribj = (jnp.where(present, var_loss, 0.0)
                        + jnp.where(jnp.logical_and(present, bg_present),
                                    w * sepj, 0.0))
            contrib = jnp.where(rowidx == 0, 0.0, contribj)
            loss_b = jnp.sum(contrib) + jnp.sum(contrib0)
            ctv = jnp.where(rowidx == 0,
                            jnp.broadcast_to(use0.astype(jnp.float32),
                                             (_NSEG, 1)),
                            present.astype(jnp.float32))
            ct = jnp.maximum(jnp.sum(ctv), 1.0)
            total += loss_b / ct
        out_ref[...] = total / float(nbatch)


def kernel(prediction, target, no_bg):
    prediction = prediction.astype(jnp.float32)
    B, C, H, W = prediction.shape
    HW = H * W
    cols = HW // _NTILES
    nrb = cols // _CBLK
    pred2 = prediction.reshape(B * C, HW)
    tgt2 = target.astype(jnp.int32).reshape(B * C, HW)

    stats_raw, compact, counts_raw = _sc_stats_compact(pred2, tgt2, B, HW)
    stats_flat = stats_raw.reshape(_NTILES, B * 5 * _NSEG)
    counts = counts_raw[:, :, 0].T
    nobg = no_bg.astype(jnp.int32).reshape(B, 1)

    tiles_per_step = 2
    nsteps = _NTILES // tiles_per_step
    span = tiles_per_step * (HW // _NTILES)
    out = pl.pallas_call(
        functools.partial(_tc_body, nbatch=B, hw=HW, nrb=nrb,
                          tiles_per_step=tiles_per_step, nsteps=nsteps),
        grid=(nsteps,),
        in_specs=[
            pl.BlockSpec(memory_space=pltpu.SMEM),
            pl.BlockSpec(memory_space=pltpu.SMEM),
            pl.BlockSpec((_NTILES, B * 5 * _NSEG), lambda g: (0, 0)),
            pl.BlockSpec((B * C, span), lambda g: (0, g)),
            pl.BlockSpec((B * C, span), lambda g: (0, g)),
        ],
        out_specs=pl.BlockSpec((1, 1), lambda g: (0, 0)),
        out_shape=jax.ShapeDtypeStruct((1, 1), jnp.float32),
        scratch_shapes=[
            pltpu.VMEM((B, _NSEG, 8), jnp.float32),
            pltpu.VMEM((B, _NSEG, 8), jnp.float32),
        ],
    )(counts, nobg, stats_flat, pred2, compact)
    return out[0, 0]


# speedup vs baseline: 2.2470x; 1.1768x over previous
import functools

import jax
import jax.numpy as jnp
from jax import lax
from jax.experimental import pallas as pl
from jax.experimental.pallas import tpu as pltpu
from jax.experimental.pallas import tpu_sc as plsc

_LAMBDA = 300.0
_NSEG = 64
_NTILES = 32
_LANES = 16
_CBLK = 512
_SENTINEL = 1.0e6


def _huber(x):
    ax = jnp.abs(x)
    return jnp.where(ax < 1.0, 0.5 * x * x, ax - 0.5)


def _sc_body(pred_hbm, id_hbm, stats_hbm, compact_hbm, counts_hbm,
             ps0, ps1, ps2, ts0, h_cnt, h_p0, h_p1, h_p2, h_hub,
             cb0, cb1, cb2, stats_loc, cnt_loc, *, nbatch, hw):
    cols = hw // _NTILES
    wid = lax.axis_index("s") * 2 + lax.axis_index("c")
    lane = lax.broadcasted_iota(jnp.int32, (_LANES,), 0)
    onesf = jnp.ones((_LANES,), jnp.float32)
    zerof = jnp.zeros((_LANES,), jnp.float32)
    sentf = jnp.full((_LANES,), _SENTINEL, jnp.float32)

    for b in range(nbatch):
        def _zero(k, _):
            s = pl.ds(k * _LANES, _LANES)
            h_cnt[s] = zerof
            h_p0[s] = zerof
            h_p1[s] = zerof
            h_p2[s] = zerof
            h_hub[s] = zerof
            return 0

        lax.fori_loop(0, (_NSEG * _LANES) // _LANES, _zero, 0)

        col0 = wid * cols
        for c, pst in enumerate([ps0, ps1, ps2]):
            pltpu.sync_copy(pred_hbm.at[3 * b + c, pl.ds(col0, cols)], pst)
        pltpu.sync_copy(id_hbm.at[b, pl.ds(col0, cols)], ts0)

        def _vec(i, off):
            s = pl.ds(i * _LANES, _LANES)
            idv = ts0[s]
            addr = lane * _NSEG + idv
            p0 = ps0[s]
            p1 = ps1[s]
            p2 = ps2[s]
            plsc.addupdate_scatter(h_cnt, [addr], onesf)
            plsc.addupdate_scatter(h_p0, [addr], p0)
            plsc.addupdate_scatter(h_p1, [addr], p1)
            plsc.addupdate_scatter(h_p2, [addr], p2)
            hz = _huber(p0) + _huber(p1) + _huber(p2)
            hf = (_huber(p0 - 255.0) + _huber(p1 - 255.0)
                  + _huber(p2 - 255.0))
            plsc.addupdate_scatter(h_hub, [addr],
                                   jnp.where(idv == 0, hz, hf))
            msk = idv == 0
            plsc.store_compressed(cb0.at[pl.ds(off, _LANES)], p0, mask=msk)
            plsc.store_compressed(cb1.at[pl.ds(off, _LANES)], p1, mask=msk)
            plsc.store_compressed(cb2.at[pl.ds(off, _LANES)], p2, mask=msk)
            npix = plsc.all_reduce_population_count(msk)
            return off + jnp.max(npix)

        off = lax.fori_loop(0, cols // _LANES, _vec, jnp.int32(0))

        hists = [h_cnt, h_p0, h_p1, h_p2, h_hub]
        for st in range(5):
            for g in range(_NSEG // _LANES):
                acc = hists[st][pl.ds(g * _LANES, _LANES)]
                for l in range(1, _LANES):
                    acc = acc + hists[st][pl.ds(l * _NSEG + g * _LANES,
                                                _LANES)]
                stats_loc[pl.ds(st * _NSEG + g * _LANES, _LANES)] = acc
        pltpu.sync_copy(stats_loc, stats_hbm.at[wid, b])

        nch = lax.div(off + (_CBLK - 1), _CBLK)
        pc = nch * _CBLK

        def _fill(k, _):
            cb0[pl.ds(off + k * _LANES, _LANES)] = sentf
            cb1[pl.ds(off + k * _LANES, _LANES)] = sentf
            cb2[pl.ds(off + k * _LANES, _LANES)] = sentf
            return 0

        lax.fori_loop(0, lax.div(pc - off + (_LANES - 1), _LANES), _fill, 0)
        cnt_loc[...] = jnp.where(lane == 0, jnp.full((_LANES,), pc,
                                                     jnp.int32), 0)
        pltpu.sync_copy(cnt_loc, counts_hbm.at[wid, b])

        def _wr(jc, _):
            src = pl.ds(jc * _CBLK, _CBLK)
            dst = pl.ds(wid * cols + jc * _CBLK, _CBLK)
            pltpu.sync_copy(cb0.at[src], compact_hbm.at[3 * b + 0, dst])
            pltpu.sync_copy(cb1.at[src], compact_hbm.at[3 * b + 1, dst])
            pltpu.sync_copy(cb2.at[src], compact_hbm.at[3 * b + 2, dst])
            return 0

        lax.fori_loop(0, nch, _wr, 0)


def _sc_stats_compact(pred2, id2, nbatch, hw):
    cols = hw // _NTILES
    mesh = plsc.VectorSubcoreMesh(core_axis_name="c", subcore_axis_name="s")
    k = functools.partial(
        pl.kernel,
        out_type=[
            jax.ShapeDtypeStruct((_NTILES, nbatch, 5 * _NSEG), jnp.float32),
            jax.ShapeDtypeStruct((3 * nbatch, hw), jnp.float32),
            jax.ShapeDtypeStruct((_NTILES, nbatch, _LANES), jnp.int32),
        ],
        mesh=mesh,
        compiler_params=pltpu.CompilerParams(needs_layout_passes=False),
        scratch_types=[
            pltpu.VMEM((cols,), jnp.float32),
            pltpu.VMEM((cols,), jnp.float32),
            pltpu.VMEM((cols,), jnp.float32),
            pltpu.VMEM((cols,), jnp.int32),
            pltpu.VMEM((_NSEG * _LANES,), jnp.float32),
            pltpu.VMEM((_NSEG * _LANES,), jnp.float32),
            pltpu.VMEM((_NSEG * _LANES,), jnp.float32),
            pltpu.VMEM((_NSEG * _LANES,), jnp.float32),
            pltpu.VMEM((_NSEG * _LANES,), jnp.float32),
            pltpu.VMEM((cols + _LANES,), jnp.float32),
            pltpu.VMEM((cols + _LANES,), jnp.float32),
            pltpu.VMEM((cols + _LANES,), jnp.float32),
            pltpu.VMEM((5 * _NSEG,), jnp.float32),
            pltpu.VMEM((_LANES,), jnp.int32),
        ],
    )(functools.partial(_sc_body, nbatch=nbatch, hw=hw))
    return k(pred2, id2)


def _tc_body(counts_ref, nobg_ref, stats_ref, pred_ref, comp_ref, out_ref,
             seg_ref, acc_ref, *, nbatch, hw, nrb, tiles_per_step, nsteps):
    g = pl.program_id(0)
    cols = hw // _NTILES

    @pl.when(g == 0)
    def _init():
        acc_ref[...] = jnp.zeros_like(acc_ref)
        stats1 = lax.dot_general(
            jnp.ones((1, _NTILES), jnp.float32), stats_ref[...],
            (((1,), (0,)), ((), ())),
            preferred_element_type=jnp.float32)
        eye = (lax.broadcasted_iota(jnp.int32, (_NSEG, _NSEG), 0)
               == lax.broadcasted_iota(jnp.int32, (_NSEG, _NSEG), 1)
               ).astype(jnp.float32)
        for bb in range(nbatch):
            base = bb * 5 * _NSEG
            cols = []
            for st in range(5):
                row = stats1[:, base + st * _NSEG:base + (st + 1) * _NSEG]
                cols.append(lax.dot_general(
                    eye, row, (((1,), (1,)), ((), ())),
                    preferred_element_type=jnp.float32))
            cnt, sp0, sp1, sp2, hub = cols
            size_safe = jnp.maximum(cnt, 1.0)
            means = jnp.concatenate([sp0, sp1, sp2], axis=1) / size_safe
            mnorm = jnp.sum(means * means, axis=1, keepdims=True)
            seg_ref[bb] = jnp.concatenate(
                [means, mnorm, cnt, hub, jnp.zeros((_NSEG, 2), jnp.float32)],
                axis=1)

    pred_all = pred_ref[...]

    for bb in range(nbatch):
        seg = seg_ref[bb]
        pred = pred_all[3 * bb:3 * bb + 3, :]

        m0 = seg[0:1, 0:3]
        mn0 = seg[0:1, 3:4]
        pn = jnp.sum(pred * pred, axis=0, keepdims=True)
        g0 = lax.dot_general(m0, pred, (((1,), (0,)), ((), ())),
                             preferred_element_type=jnp.float32)
        t0 = _LAMBDA / (1.0 + mn0 + pn - 2.0 * g0)
        acc_ref[bb, 0:1, 1:2] += jnp.sum(t0).reshape(1, 1)

        means = seg[:, 0:3]
        mn = seg[:, 3:4]
        for r in range(tiles_per_step):
            pc = counts_ref[bb, g * tiles_per_step + r]
            base = r * cols

            def _chunk(jc, _, bb=bb, means=means, mn=mn, base=base):
                comp = comp_ref[pl.ds(3 * bb, 3),
                                pl.ds(base + jc * _CBLK, _CBLK)]
                cpn = jnp.sum(comp * comp, axis=0, keepdims=True)
                gg = lax.dot_general(means, comp, (((1,), (0,)), ((), ())),
                                     preferred_element_type=jnp.float32)
                tm = _LAMBDA / (1.0 + mn + cpn - 2.0 * gg)
                acc_ref[bb, :, 0:1] += jnp.sum(tm, axis=1, keepdims=True)
                return 0

            lax.fori_loop(0, pc // _CBLK, _chunk, 0)

    @pl.when(g == nsteps - 1)
    def _finalize():
        total = jnp.zeros((1, 1), dtype=jnp.float32)
        rowidx = lax.broadcasted_iota(jnp.int32, (_NSEG, 1), 0)
        for bb in range(nbatch):
            segb = seg_ref[bb]
            cnt = segb[:, 4:5]
            hub = segb[:, 5:6]
            S = acc_ref[bb, :, 0:1]
            rs0 = acc_ref[bb, 0:1, 1:2]
            nobg = nobg_ref[bb, 0]
            present = cnt > 0.0
            size_safe = jnp.maximum(cnt, 1.0)
            var_loss = hub / (size_safe * 3.0)
            w = 10.0 * lax.rsqrt(size_safe)
            cnt0 = cnt[0:1, :]
            bg_present = cnt0 > 0.0
            use0 = jnp.logical_and(bg_present, nobg == 0)
            n_non = float(hw) - cnt0
            sep0 = (rs0 - S[0:1, :]) / jnp.maximum(n_non, 1.0)
            contrib0 = (jnp.where(use0, var_loss[0:1, :], 0.0)
                        + jnp.where(jnp.logical_and(use0, n_non > 0.0),
                                    w[0:1, :] * sep0, 0.0))
            sepj = S / jnp.maximum(cnt0, 1.0)
            contribj = (jnp.where(present, var_loss, 0.0)
                        + jnp.where(jnp.logical_and(present, bg_present),
                                    w * sepj, 0.0))
            contrib = jnp.where(rowidx == 0, 0.0, contribj)
            loss_b = jnp.sum(contrib) + jnp.sum(contrib0)
            ctv = jnp.where(rowidx == 0,
                            jnp.broadcast_to(use0.astype(jnp.float32),
                                             (_NSEG, 1)),
                            present.astype(jnp.float32))
            ct = jnp.maximum(jnp.sum(ctv), 1.0)
            total += loss_b / ct
        out_ref[...] = total / float(nbatch)


def kernel(prediction, target, no_bg):
    prediction = prediction.astype(jnp.float32)
    B, C, H, W = prediction.shape
    HW = H * W
    cols = HW // _NTILES
    nrb = cols // _CBLK
    pred2 = prediction.reshape(B * C, HW)
    tgt = target.astype(jnp.int32)
    id2 = (tgt[:, 0] * 16 + tgt[:, 1] * 4 + tgt[:, 2]).reshape(B, HW)

    stats_raw, compact, counts_raw = _sc_stats_compact(pred2, id2, B, HW)
    stats_flat = stats_raw.reshape(_NTILES, B * 5 * _NSEG)
    counts = counts_raw[:, :, 0].T
    nobg = no_bg.astype(jnp.int32).reshape(B, 1)

    tiles_per_step = 2
    nsteps = _NTILES // tiles_per_step
    span = tiles_per_step * (HW // _NTILES)
    out = pl.pallas_call(
        functools.partial(_tc_body, nbatch=B, hw=HW, nrb=nrb,
                          tiles_per_step=tiles_per_step, nsteps=nsteps),
        grid=(nsteps,),
        in_specs=[
            pl.BlockSpec(memory_space=pltpu.SMEM),
            pl.BlockSpec(memory_space=pltpu.SMEM),
            pl.BlockSpec((_NTILES, B * 5 * _NSEG), lambda g: (0, 0)),
            pl.BlockSpec((B * C, span), lambda g: (0, g)),
            pl.BlockSpec((B * C, span), lambda g: (0, g)),
        ],
        out_specs=pl.BlockSpec((1, 1), lambda g: (0, 0)),
        out_shape=jax.ShapeDtypeStruct((1, 1), jnp.float32),
        scratch_shapes=[
            pltpu.VMEM((B, _NSEG, 8), jnp.float32),
            pltpu.VMEM((B, _NSEG, 8), jnp.float32),
        ],
    )(counts, nobg, stats_flat, pred2, compact)
    return out[0, 0]


# speedup vs baseline: 2.3659x; 1.0529x over previous
import functools

import jax
import jax.numpy as jnp
from jax import lax
from jax.experimental import pallas as pl
from jax.experimental.pallas import tpu as pltpu
from jax.experimental.pallas import tpu_sc as plsc

_LAMBDA = 300.0
_NSEG = 64
_NTILES = 32
_LANES = 16
_CBLK = 512
_SENTINEL = 1.0e6


def _huber(x):
    ax = jnp.abs(x)
    return jnp.where(ax < 1.0, 0.5 * x * x, ax - 0.5)


def _sc_body(pred_hbm, id_hbm, stats_hbm, compact_hbm, counts_hbm,
             ps0, ps1, ps2, ts0, h_cnt, h_p0, h_p1, h_p2, h_hub,
             cb0, cb1, cb2, stats_loc, cnt_loc, *, nbatch, hw):
    cols = hw // _NTILES
    wid = lax.axis_index("s") * 2 + lax.axis_index("c")
    lane = lax.broadcasted_iota(jnp.int32, (_LANES,), 0)
    onesf = jnp.ones((_LANES,), jnp.float32)
    zerof = jnp.zeros((_LANES,), jnp.float32)
    sentf = jnp.full((_LANES,), _SENTINEL, jnp.float32)

    for b in range(nbatch):
        def _zero(k, _):
            s = pl.ds(k * _LANES, _LANES)
            h_cnt[s] = zerof
            h_p0[s] = zerof
            h_p1[s] = zerof
            h_p2[s] = zerof
            h_hub[s] = zerof
            return 0

        lax.fori_loop(0, (_NSEG * _LANES) // _LANES, _zero, 0)

        col0 = wid * cols
        for c, pst in enumerate([ps0, ps1, ps2]):
            pltpu.sync_copy(pred_hbm.at[3 * b + c, pl.ds(col0, cols)], pst)
        pltpu.sync_copy(id_hbm.at[b, pl.ds(col0, cols)], ts0)

        def _vec(i, ol):
            s = pl.ds(i * _LANES, _LANES)
            idv = ts0[s]
            addr = lane * _NSEG + idv
            p0 = ps0[s]
            p1 = ps1[s]
            p2 = ps2[s]
            plsc.addupdate_scatter(h_cnt, [addr], onesf)
            plsc.addupdate_scatter(h_p0, [addr], p0)
            plsc.addupdate_scatter(h_p1, [addr], p1)
            plsc.addupdate_scatter(h_p2, [addr], p2)
            hz = _huber(p0) + _huber(p1) + _huber(p2)
            hf = (_huber(p0 - 255.0) + _huber(p1 - 255.0)
                  + _huber(p2 - 255.0))
            plsc.addupdate_scatter(h_hub, [addr],
                                   jnp.where(idv == 0, hz, hf))
            msk = idv == 0
            cidx = ol * _LANES + lane
            plsc.store_scatter(cb0, [cidx], p0, mask=msk)
            plsc.store_scatter(cb1, [cidx], p1, mask=msk)
            plsc.store_scatter(cb2, [cidx], p2, mask=msk)
            return ol + jnp.where(msk, 1, 0)

        ol = lax.fori_loop(0, cols // _LANES, _vec,
                           jnp.zeros((_LANES,), jnp.int32))
        off = jnp.max(ol) * _LANES

        hists = [h_cnt, h_p0, h_p1, h_p2, h_hub]
        for st in range(5):
            for g in range(_NSEG // _LANES):
                acc = hists[st][pl.ds(g * _LANES, _LANES)]
                for l in range(1, _LANES):
                    acc = acc + hists[st][pl.ds(l * _NSEG + g * _LANES,
                                                _LANES)]
                stats_loc[pl.ds(st * _NSEG + g * _LANES, _LANES)] = acc
        pltpu.sync_copy(stats_loc, stats_hbm.at[wid, b])

        nch = lax.div(off + (_CBLK - 1), _CBLK)
        pc = nch * _CBLK

        def _fill(k, _):
            fidx = k * _LANES + lane
            fmsk = k >= ol
            plsc.store_scatter(cb0, [fidx], sentf, mask=fmsk)
            plsc.store_scatter(cb1, [fidx], sentf, mask=fmsk)
            plsc.store_scatter(cb2, [fidx], sentf, mask=fmsk)
            return 0

        lax.fori_loop(0, lax.div(pc, _LANES), _fill, 0)
        cnt_loc[...] = jnp.where(lane == 0, jnp.full((_LANES,), pc,
                                                     jnp.int32), 0)
        pltpu.sync_copy(cnt_loc, counts_hbm.at[wid, b])

        def _wr(jc, _):
            src = pl.ds(jc * _CBLK, _CBLK)
            dst = pl.ds(wid * cols + jc * _CBLK, _CBLK)
            pltpu.sync_copy(cb0.at[src], compact_hbm.at[3 * b + 0, dst])
            pltpu.sync_copy(cb1.at[src], compact_hbm.at[3 * b + 1, dst])
            pltpu.sync_copy(cb2.at[src], compact_hbm.at[3 * b + 2, dst])
            return 0

        lax.fori_loop(0, nch, _wr, 0)


def _sc_stats_compact(pred2, id2, nbatch, hw):
    cols = hw // _NTILES
    mesh = plsc.VectorSubcoreMesh(core_axis_name="c", subcore_axis_name="s")
    k = functools.partial(
        pl.kernel,
        out_type=[
            jax.ShapeDtypeStruct((_NTILES, nbatch, 5 * _NSEG), jnp.float32),
            jax.ShapeDtypeStruct((3 * nbatch, hw), jnp.float32),
            jax.ShapeDtypeStruct((_NTILES, nbatch, _LANES), jnp.int32),
        ],
        mesh=mesh,
        compiler_params=pltpu.CompilerParams(needs_layout_passes=False),
        scratch_types=[
            pltpu.VMEM((cols,), jnp.float32),
            pltpu.VMEM((cols,), jnp.float32),
            pltpu.VMEM((cols,), jnp.float32),
            pltpu.VMEM((cols,), jnp.int32),
            pltpu.VMEM((_NSEG * _LANES,), jnp.float32),
            pltpu.VMEM((_NSEG * _LANES,), jnp.float32),
            pltpu.VMEM((_NSEG * _LANES,), jnp.float32),
            pltpu.VMEM((_NSEG * _LANES,), jnp.float32),
            pltpu.VMEM((_NSEG * _LANES,), jnp.float32),
            pltpu.VMEM((cols + _LANES,), jnp.float32),
            pltpu.VMEM((cols + _LANES,), jnp.float32),
            pltpu.VMEM((cols + _LANES,), jnp.float32),
            pltpu.VMEM((5 * _NSEG,), jnp.float32),
            pltpu.VMEM((_LANES,), jnp.int32),
        ],
    )(functools.partial(_sc_body, nbatch=nbatch, hw=hw))
    return k(pred2, id2)


def _tc_body(counts_ref, nobg_ref, stats_ref, pred_ref, comp_ref, out_ref,
             seg_ref, acc_ref, *, nbatch, hw, nrb, tiles_per_step, nsteps):
    g = pl.program_id(0)
    cols = hw // _NTILES

    @pl.when(g == 0)
    def _init():
        acc_ref[...] = jnp.zeros_like(acc_ref)
        stats1 = lax.dot_general(
            jnp.ones((1, _NTILES), jnp.float32), stats_ref[...],
            (((1,), (0,)), ((), ())),
            preferred_element_type=jnp.float32)
        eye = (lax.broadcasted_iota(jnp.int32, (_NSEG, _NSEG), 0)
               == lax.broadcasted_iota(jnp.int32, (_NSEG, _NSEG), 1)
               ).astype(jnp.float32)
        for bb in range(nbatch):
            base = bb * 5 * _NSEG
            cols = []
            for st in range(5):
                row = stats1[:, base + st * _NSEG:base + (st + 1) * _NSEG]
                cols.append(lax.dot_general(
                    eye, row, (((1,), (1,)), ((), ())),
                    preferred_element_type=jnp.float32))
            cnt, sp0, sp1, sp2, hub = cols
            size_safe = jnp.maximum(cnt, 1.0)
            means = jnp.concatenate([sp0, sp1, sp2], axis=1) / size_safe
            mnorm = jnp.sum(means * means, axis=1, keepdims=True)
            seg_ref[bb] = jnp.concatenate(
                [means, mnorm, cnt, hub, jnp.zeros((_NSEG, 2), jnp.float32)],
                axis=1)

    pred_all = pred_ref[...]

    for bb in range(nbatch):
        seg = seg_ref[bb]
        pred = pred_all[3 * bb:3 * bb + 3, :]

        m0 = seg[0:1, 0:3]
        mn0 = seg[0:1, 3:4]
        pn = jnp.sum(pred * pred, axis=0, keepdims=True)
        g0 = lax.dot_general(m0, pred, (((1,), (0,)), ((), ())),
                             preferred_element_type=jnp.float32)
        t0 = _LAMBDA / (1.0 + mn0 + pn - 2.0 * g0)
        acc_ref[bb, 0:1, 1:2] += jnp.sum(t0).reshape(1, 1)

        means = seg[:, 0:3]
        mn = seg[:, 3:4]
        for r in range(tiles_per_step):
            pc = counts_ref[bb, g * tiles_per_step + r]
            base = r * cols

            def _chunk(jc, _, bb=bb, means=means, mn=mn, base=base):
                comp = comp_ref[pl.ds(3 * bb, 3),
                                pl.ds(base + jc * _CBLK, _CBLK)]
                cpn = jnp.sum(comp * comp, axis=0, keepdims=True)
                gg = lax.dot_general(means, comp, (((1,), (0,)), ((), ())),
                                     preferred_element_type=jnp.float32)
                tm = _LAMBDA / (1.0 + mn + cpn - 2.0 * gg)
                acc_ref[bb, :, 0:1] += jnp.sum(tm, axis=1, keepdims=True)
                return 0

            lax.fori_loop(0, pc // _CBLK, _chunk, 0)

    @pl.when(g == nsteps - 1)
    def _finalize():
        total = jnp.zeros((1, 1), dtype=jnp.float32)
        rowidx = lax.broadcasted_iota(jnp.int32, (_NSEG, 1), 0)
        for bb in range(nbatch):
            segb = seg_ref[bb]
            cnt = segb[:, 4:5]
            hub = segb[:, 5:6]
            S = acc_ref[bb, :, 0:1]
            rs0 = acc_ref[bb, 0:1, 1:2]
            nobg = nobg_ref[bb, 0]
            present = cnt > 0.0
            size_safe = jnp.maximum(cnt, 1.0)
            var_loss = hub / (size_safe * 3.0)
            w = 10.0 * lax.rsqrt(size_safe)
            cnt0 = cnt[0:1, :]
            bg_present = cnt0 > 0.0
            use0 = jnp.logical_and(bg_present, nobg == 0)
            n_non = float(hw) - cnt0
            sep0 = (rs0 - S[0:1, :]) / jnp.maximum(n_non, 1.0)
            contrib0 = (jnp.where(use0, var_loss[0:1, :], 0.0)
                        + jnp.where(jnp.logical_and(use0, n_non > 0.0),
                                    w[0:1, :] * sep0, 0.0))
            sepj = S / jnp.maximum(cnt0, 1.0)
            contribj = (jnp.where(present, var_loss, 0.0)
                        + jnp.where(jnp.logical_and(present, bg_present),
                                    w * sepj, 0.0))
            contrib = jnp.where(rowidx == 0, 0.0, contribj)
            loss_b = jnp.sum(contrib) + jnp.sum(contrib0)
            ctv = jnp.where(rowidx == 0,
                            jnp.broadcast_to(use0.astype(jnp.float32),
                                             (_NSEG, 1)),
                            present.astype(jnp.float32))
            ct = jnp.maximum(jnp.sum(ctv), 1.0)
            total += loss_b / ct
        out_ref[...] = total / float(nbatch)


def kernel(prediction, target, no_bg):
    prediction = prediction.astype(jnp.float32)
    B, C, H, W = prediction.shape
    HW = H * W
    cols = HW // _NTILES
    nrb = cols // _CBLK
    pred2 = prediction.reshape(B * C, HW)
    tgt = target.astype(jnp.int32)
    id2 = (tgt[:, 0] * 16 + tgt[:, 1] * 4 + tgt[:, 2]).reshape(B, HW)

    stats_raw, compact, counts_raw = _sc_stats_compact(pred2, id2, B, HW)
    stats_flat = stats_raw.reshape(_NTILES, B * 5 * _NSEG)
    counts = counts_raw[:, :, 0].T
    nobg = no_bg.astype(jnp.int32).reshape(B, 1)

    tiles_per_step = 4
    nsteps = _NTILES // tiles_per_step
    span = tiles_per_step * (HW // _NTILES)
    out = pl.pallas_call(
        functools.partial(_tc_body, nbatch=B, hw=HW, nrb=nrb,
                          tiles_per_step=tiles_per_step, nsteps=nsteps),
        grid=(nsteps,),
        in_specs=[
            pl.BlockSpec(memory_space=pltpu.SMEM),
            pl.BlockSpec(memory_space=pltpu.SMEM),
            pl.BlockSpec((_NTILES, B * 5 * _NSEG), lambda g: (0, 0)),
            pl.BlockSpec((B * C, span), lambda g: (0, g)),
            pl.BlockSpec((B * C, span), lambda g: (0, g)),
        ],
        out_specs=pl.BlockSpec((1, 1), lambda g: (0, 0)),
        out_shape=jax.ShapeDtypeStruct((1, 1), jnp.float32),
        scratch_shapes=[
            pltpu.VMEM((B, _NSEG, 8), jnp.float32),
            pltpu.VMEM((B, _NSEG, 8), jnp.float32),
        ],
    )(counts, nobg, stats_flat, pred2, compact)
    return out[0, 0]


# speedup vs baseline: 2.3840x; 1.0077x over previous
import functools

import jax
import jax.numpy as jnp
from jax import lax
from jax.experimental import pallas as pl
from jax.experimental.pallas import tpu as pltpu
from jax.experimental.pallas import tpu_sc as plsc

_LAMBDA = 300.0
_NSEG = 64
_NTILES = 32
_LANES = 16
_CBLK = 512
_SENTINEL = 1.0e6


def _huber(x):
    ax = jnp.abs(x)
    return jnp.where(ax < 1.0, 0.5 * x * x, ax - 0.5)


def _sc_body(pred_hbm, id_hbm, stats_hbm, compact_hbm, counts_hbm,
             ps0, ps1, ps2, ts0, h_cnt, h_p0, h_p1, h_p2, h_hub,
             cb0, cb1, cb2, stats_loc, cnt_loc, *, nbatch, hw):
    cols = hw // _NTILES
    wid = lax.axis_index("s") * 2 + lax.axis_index("c")
    lane = lax.broadcasted_iota(jnp.int32, (_LANES,), 0)
    onesf = jnp.ones((_LANES,), jnp.float32)
    zerof = jnp.zeros((_LANES,), jnp.float32)
    sentf = jnp.full((_LANES,), _SENTINEL, jnp.float32)

    for b in range(nbatch):
        def _zero(k, _):
            s = pl.ds(k * _LANES, _LANES)
            h_cnt[s] = zerof
            h_p0[s] = zerof
            h_p1[s] = zerof
            h_p2[s] = zerof
            h_hub[s] = zerof
            return 0

        lax.fori_loop(0, (_NSEG * _LANES) // _LANES, _zero, 0)

        col0 = wid * cols
        for c, pst in enumerate([ps0, ps1, ps2]):
            pltpu.sync_copy(pred_hbm.at[3 * b + c, pl.ds(col0, cols)], pst)
        pltpu.sync_copy(id_hbm.at[b, pl.ds(col0, cols)], ts0)

        def _vec(i, ol):
            s = pl.ds(i * _LANES, _LANES)
            idv = ts0[s]
            addr = lane * _NSEG + idv
            p0 = ps0[s]
            p1 = ps1[s]
            p2 = ps2[s]
            plsc.addupdate_scatter(h_cnt, [addr], onesf)
            plsc.addupdate_scatter(h_p0, [addr], p0)
            plsc.addupdate_scatter(h_p1, [addr], p1)
            plsc.addupdate_scatter(h_p2, [addr], p2)
            hz = _huber(p0) + _huber(p1) + _huber(p2)
            hf = (_huber(p0 - 255.0) + _huber(p1 - 255.0)
                  + _huber(p2 - 255.0))
            plsc.addupdate_scatter(h_hub, [addr],
                                   jnp.where(idv == 0, hz, hf))
            msk = idv == 0
            cidx = ol * _LANES + lane
            plsc.store_scatter(cb0, [cidx], p0, mask=msk)
            plsc.store_scatter(cb1, [cidx], p1, mask=msk)
            plsc.store_scatter(cb2, [cidx], p2, mask=msk)
            return ol + jnp.where(msk, 1, 0)

        ol = lax.fori_loop(0, cols // _LANES, _vec,
                           jnp.zeros((_LANES,), jnp.int32))
        off = jnp.max(ol) * _LANES

        hists = [h_cnt, h_p0, h_p1, h_p2, h_hub]
        for st in range(5):
            for g in range(_NSEG // _LANES):
                acc = hists[st][pl.ds(g * _LANES, _LANES)]
                for l in range(1, _LANES):
                    acc = acc + hists[st][pl.ds(l * _NSEG + g * _LANES,
                                                _LANES)]
                stats_loc[pl.ds(st * _NSEG + g * _LANES, _LANES)] = acc
        pltpu.sync_copy(stats_loc, stats_hbm.at[wid, b])

        nch = lax.div(off + (_CBLK - 1), _CBLK)
        pc = nch * _CBLK

        def _fill(k, _):
            fidx = k * _LANES + lane
            fmsk = k >= ol
            plsc.store_scatter(cb0, [fidx], sentf, mask=fmsk)
            plsc.store_scatter(cb1, [fidx], sentf, mask=fmsk)
            plsc.store_scatter(cb2, [fidx], sentf, mask=fmsk)
            return 0

        lax.fori_loop(0, lax.div(pc, _LANES), _fill, 0)
        cnt_loc[...] = jnp.where(lane == 0, jnp.full((_LANES,), pc,
                                                     jnp.int32), 0)
        pltpu.sync_copy(cnt_loc, counts_hbm.at[wid, b])

        def _wr(jc, _):
            src = pl.ds(jc * _CBLK, _CBLK)
            dst = pl.ds(wid * cols + jc * _CBLK, _CBLK)
            pltpu.sync_copy(cb0.at[src], compact_hbm.at[3 * b + 0, dst])
            pltpu.sync_copy(cb1.at[src], compact_hbm.at[3 * b + 1, dst])
            pltpu.sync_copy(cb2.at[src], compact_hbm.at[3 * b + 2, dst])
            return 0

        lax.fori_loop(0, nch, _wr, 0)


def _sc_stats_compact(pred2, id2, nbatch, hw):
    cols = hw // _NTILES
    mesh = plsc.VectorSubcoreMesh(core_axis_name="c", subcore_axis_name="s")
    k = functools.partial(
        pl.kernel,
        out_type=[
            jax.ShapeDtypeStruct((_NTILES, nbatch, 5 * _NSEG), jnp.float32),
            jax.ShapeDtypeStruct((3 * nbatch, hw), jnp.float32),
            jax.ShapeDtypeStruct((_NTILES, nbatch, _LANES), jnp.int32),
        ],
        mesh=mesh,
        compiler_params=pltpu.CompilerParams(needs_layout_passes=False),
        scratch_types=[
            pltpu.VMEM((cols,), jnp.float32),
            pltpu.VMEM((cols,), jnp.float32),
            pltpu.VMEM((cols,), jnp.float32),
            pltpu.VMEM((cols,), jnp.int32),
            pltpu.VMEM((_NSEG * _LANES,), jnp.float32),
            pltpu.VMEM((_NSEG * _LANES,), jnp.float32),
            pltpu.VMEM((_NSEG * _LANES,), jnp.float32),
            pltpu.VMEM((_NSEG * _LANES,), jnp.float32),
            pltpu.VMEM((_NSEG * _LANES,), jnp.float32),
            pltpu.VMEM((cols + _LANES,), jnp.float32),
            pltpu.VMEM((cols + _LANES,), jnp.float32),
            pltpu.VMEM((cols + _LANES,), jnp.float32),
            pltpu.VMEM((5 * _NSEG,), jnp.float32),
            pltpu.VMEM((_LANES,), jnp.int32),
        ],
    )(functools.partial(_sc_body, nbatch=nbatch, hw=hw))
    return k(pred2, id2)


def _tc_body(counts_ref, nobg_ref, stats_ref, pred_ref, comp_ref, out_ref,
             seg_ref, acc_ref, *, nbatch, hw, nrb, tiles_per_step, nsteps):
    g = pl.program_id(0)
    cols = hw // _NTILES

    @pl.when(g == 0)
    def _init():
        acc_ref[...] = jnp.zeros_like(acc_ref)
        stats1 = lax.dot_general(
            jnp.ones((1, _NTILES), jnp.float32), stats_ref[...],
            (((1,), (0,)), ((), ())),
            preferred_element_type=jnp.float32)
        eye = (lax.broadcasted_iota(jnp.int32, (_NSEG, _NSEG), 0)
               == lax.broadcasted_iota(jnp.int32, (_NSEG, _NSEG), 1)
               ).astype(jnp.float32)
        for bb in range(nbatch):
            base = bb * 5 * _NSEG
            cols = []
            for st in range(5):
                row = stats1[:, base + st * _NSEG:base + (st + 1) * _NSEG]
                cols.append(lax.dot_general(
                    eye, row, (((1,), (1,)), ((), ())),
                    preferred_element_type=jnp.float32))
            cnt, sp0, sp1, sp2, hub = cols
            size_safe = jnp.maximum(cnt, 1.0)
            means = jnp.concatenate([sp0, sp1, sp2], axis=1) / size_safe
            mnorm = jnp.sum(means * means, axis=1, keepdims=True)
            seg_ref[bb] = jnp.concatenate(
                [means, mnorm, cnt, hub, jnp.zeros((_NSEG, 2), jnp.float32)],
                axis=1)

    pred_all = pred_ref[...]

    for bb in range(nbatch):
        seg = seg_ref[bb]
        pred = pred_all[3 * bb:3 * bb + 3, :]

        m0 = seg[0:1, 0:3]
        mn0 = seg[0:1, 3:4]
        pn = jnp.sum(pred * pred, axis=0, keepdims=True)
        g0 = lax.dot_general(m0, pred, (((1,), (0,)), ((), ())),
                             preferred_element_type=jnp.float32)
        t0 = _LAMBDA / (1.0 + mn0 + pn - 2.0 * g0)
        acc_ref[bb, 0:1, 1:2] += jnp.sum(t0).reshape(1, 1)

        means = seg[:, 0:3]
        mn = seg[:, 3:4]
        for r in range(tiles_per_step):
            pc = counts_ref[bb, g * tiles_per_step + r]
            base = r * cols

            def _chunk(jc, _, bb=bb, means=means, mn=mn, base=base):
                comp = comp_ref[pl.ds(3 * bb, 3),
                                pl.ds(base + jc * _CBLK, _CBLK)]
                cpn = jnp.sum(comp * comp, axis=0, keepdims=True)
                gg = lax.dot_general(means, comp, (((1,), (0,)), ((), ())),
                                     preferred_element_type=jnp.float32)
                tm = _LAMBDA / (1.0 + mn + cpn - 2.0 * gg)
                acc_ref[bb, :, 0:1] += jnp.sum(tm, axis=1, keepdims=True)
                return 0

            lax.fori_loop(0, pc // _CBLK, _chunk, 0)

    @pl.when(g == nsteps - 1)
    def _finalize():
        total = jnp.zeros((1, 1), dtype=jnp.float32)
        rowidx = lax.broadcasted_iota(jnp.int32, (_NSEG, 1), 0)
        for bb in range(nbatch):
            segb = seg_ref[bb]
            cnt = segb[:, 4:5]
            hub = segb[:, 5:6]
            S = acc_ref[bb, :, 0:1]
            rs0 = acc_ref[bb, 0:1, 1:2]
            nobg = nobg_ref[bb, 0]
            present = cnt > 0.0
            size_safe = jnp.maximum(cnt, 1.0)
            var_loss = hub / (size_safe * 3.0)
            w = 10.0 * lax.rsqrt(size_safe)
            cnt0 = cnt[0:1, :]
            bg_present = cnt0 > 0.0
            use0 = jnp.logical_and(bg_present, nobg == 0)
            n_non = float(hw) - cnt0
            sep0 = (rs0 - S[0:1, :]) / jnp.maximum(n_non, 1.0)
            contrib0 = (jnp.where(use0, var_loss[0:1, :], 0.0)
                        + jnp.where(jnp.logical_and(use0, n_non > 0.0),
                                    w[0:1, :] * sep0, 0.0))
            sepj = S / jnp.maximum(cnt0, 1.0)
            contribj = (jnp.where(present, var_loss, 0.0)
                        + jnp.where(jnp.logical_and(present, bg_present),
                                    w * sepj, 0.0))
            contrib = jnp.where(rowidx == 0, 0.0, contribj)
            loss_b = jnp.sum(contrib) + jnp.sum(contrib0)
            ctv = jnp.where(rowidx == 0,
                            jnp.broadcast_to(use0.astype(jnp.float32),
                                             (_NSEG, 1)),
                            present.astype(jnp.float32))
            ct = jnp.maximum(jnp.sum(ctv), 1.0)
            total += loss_b / ct
        out_ref[...] = total / float(nbatch)


def kernel(prediction, target, no_bg):
    prediction = prediction.astype(jnp.float32)
    B, C, H, W = prediction.shape
    HW = H * W
    cols = HW // _NTILES
    nrb = cols // _CBLK
    pred2 = prediction.reshape(B * C, HW)
    tgt = target.astype(jnp.int32)
    id2 = (tgt[:, 0] * 16 + tgt[:, 1] * 4 + tgt[:, 2]).reshape(B, HW)

    stats_raw, compact, counts_raw = _sc_stats_compact(pred2, id2, B, HW)
    stats_flat = stats_raw.reshape(_NTILES, B * 5 * _NSEG)
    counts = counts_raw[:, :, 0].T
    nobg = no_bg.astype(jnp.int32).reshape(B, 1)

    tiles_per_step = 8
    nsteps = _NTILES // tiles_per_step
    span = tiles_per_step * (HW // _NTILES)
    out = pl.pallas_call(
        functools.partial(_tc_body, nbatch=B, hw=HW, nrb=nrb,
                          tiles_per_step=tiles_per_step, nsteps=nsteps),
        grid=(nsteps,),
        in_specs=[
            pl.BlockSpec(memory_space=pltpu.SMEM),
            pl.BlockSpec(memory_space=pltpu.SMEM),
            pl.BlockSpec((_NTILES, B * 5 * _NSEG), lambda g: (0, 0)),
            pl.BlockSpec((B * C, span), lambda g: (0, g)),
            pl.BlockSpec((B * C, span), lambda g: (0, g)),
        ],
        out_specs=pl.BlockSpec((1, 1), lambda g: (0, 0)),
        out_shape=jax.ShapeDtypeStruct((1, 1), jnp.float32),
        scratch_shapes=[
            pltpu.VMEM((B, _NSEG, 8), jnp.float32),
            pltpu.VMEM((B, _NSEG, 8), jnp.float32),
        ],
    )(counts, nobg, stats_flat, pred2, compact)
    return out[0, 0]
